# baseline (speedup 1.0000x reference)
import functools

import jax
import jax.numpy as jnp
from jax import lax
from jax.experimental import pallas as pl
from jax.experimental.pallas import tpu as pltpu
from jax.experimental.pallas import tpu_sc as plsc

_NC = 2
_NS = 16
_L = 16
_NW = _NC * _NS


def _make_sc_add(B, S, D):
    SPW = S // _NW
    R = 8
    CH = R * D
    NCHUNK = SPW // R

    def body(x_hbm, pos_hbm, out_hbm, xbuf, pbuf, lsem, ssem):
        wid = lax.axis_index("s") * _NC + lax.axis_index("c")
        base = wid * (SPW * D)

        def chunk(ci, carry):
            off = base + ci * CH
            loads = [pltpu.async_copy(pos_hbm.at[pl.ds(off, CH)], pbuf, lsem)]
            for b in range(B):
                loads.append(
                    pltpu.async_copy(x_hbm.at[b, pl.ds(off, CH)], xbuf.at[b], lsem)
                )
            for cp in loads:
                cp.wait()

            def vec(i, c):
                ds = pl.ds(i * _L, _L)
                pv = pbuf[ds]
                for b in range(B):
                    xbuf[b, ds] = xbuf[b, ds] + pv
                return c

            lax.fori_loop(0, CH // _L, vec, 0)

            stores = [
                pltpu.async_copy(xbuf.at[b], out_hbm.at[b, pl.ds(off, CH)], ssem)
                for b in range(B)
            ]
            for cp in stores:
                cp.wait()
            return carry

        lax.fori_loop(0, NCHUNK, chunk, 0)

    mesh = plsc.VectorSubcoreMesh(core_axis_name="c", subcore_axis_name="s")
    return pl.kernel(
        body,
        out_type=jax.ShapeDtypeStruct((B, S * D), jnp.float32),
        mesh=mesh,
        scratch_types=[
            pltpu.VMEM((B, CH), jnp.float32),
            pltpu.VMEM((CH,), jnp.float32),
            pltpu.SemaphoreType.DMA,
            pltpu.SemaphoreType.DMA,
        ],
    )


def kernel(x, position_embeddings):
    B, S, D = x.shape
    pos = position_embeddings[:S].reshape(S * D)
    xf = x.reshape(B, S * D)
    out = _make_sc_add(B, S, D)(xf, pos)
    return out.reshape(B, S, D)


# speedup vs baseline: 1.2238x; 1.2238x over previous
import jax
import jax.numpy as jnp
from jax import lax
from jax.experimental import pallas as pl
from jax.experimental.pallas import tpu as pltpu
from jax.experimental.pallas import tpu_sc as plsc

_NC = 2
_NS = 16
_L = 16
_NW = _NC * _NS
_RING = 4


def _make_sc_add(B, S, D):
    SPW = S // _NW
    R = 4
    CH = R * D
    NCHUNK = SPW // R

    def body(x_hbm, pos_hbm, out_hbm, xbuf, pbuf, *sems):
        lsem = sems[:_RING]
        ssem = sems[_RING:]
        wid = lax.axis_index("s") * _NC + lax.axis_index("c")
        base = wid * (SPW * D)

        def issue_loads(cc, q):
            off = base + cc * CH
            pltpu.async_copy(pos_hbm.at[pl.ds(off, CH)], pbuf.at[q], lsem[q])
            for b in range(B):
                pltpu.async_copy(
                    x_hbm.at[b, pl.ds(off, CH)], xbuf.at[q, b], lsem[q]
                )

        def wait_loads(q):
            pltpu.make_async_copy(pos_hbm.at[pl.ds(0, CH)], pbuf.at[q], lsem[q]).wait()
            for b in range(B):
                pltpu.make_async_copy(
                    x_hbm.at[b, pl.ds(0, CH)], xbuf.at[q, b], lsem[q]
                ).wait()

        def issue_stores(cc, q):
            off = base + cc * CH
            for b in range(B):
                pltpu.async_copy(
                    xbuf.at[q, b], out_hbm.at[b, pl.ds(off, CH)], ssem[q]
                )

        def wait_stores(q):
            for b in range(B):
                pltpu.make_async_copy(
                    xbuf.at[q, b], out_hbm.at[b, pl.ds(0, CH)], ssem[q]
                ).wait()

        issue_loads(0, 0)

        @pl.loop(0, NCHUNK, step=_RING)
        def _(ci):
            for q in range(_RING):
                cc = ci + q
                nq = (q + 1) % _RING

                @pl.when(cc >= _RING - 1)
                def _():
                    wait_stores(nq)

                @pl.when(cc < NCHUNK - 1)
                def _():
                    issue_loads(cc + 1, nq)

                wait_loads(q)

                @plsc.parallel_loop(0, CH // _L, unroll=8)
                def _(i):
                    ds = pl.ds(i * _L, _L)
                    pv = pbuf[q, ds]
                    for b in range(B):
                        xbuf[q, b, ds] = xbuf[q, b, ds] + pv

                issue_stores(cc, q)

        for q in ((NCHUNK - 3) % _RING, (NCHUNK - 2) % _RING, (NCHUNK - 1) % _RING):
            wait_stores(q)

    mesh = plsc.VectorSubcoreMesh(core_axis_name="c", subcore_axis_name="s")
    return pl.kernel(
        body,
        out_type=jax.ShapeDtypeStruct((B, S * D), jnp.float32),
        mesh=mesh,
        scratch_types=(
            [
                pltpu.VMEM((_RING, B, CH), jnp.float32),
                pltpu.VMEM((_RING, CH), jnp.float32),
            ]
            + [pltpu.SemaphoreType.DMA] * (2 * _RING)
        ),
    )


def kernel(x, position_embeddings):
    B, S, D = x.shape
    pos = position_embeddings[:S].reshape(S * D)
    xf = x.reshape(B, S * D)
    out = _make_sc_add(B, S, D)(xf, pos)
    return out.reshape(B, S, D)


# speedup vs baseline: 3.6179x; 2.9563x over previous
import jax
import jax.numpy as jnp
from jax import lax
from jax.experimental import pallas as pl
from jax.experimental.pallas import tpu as pltpu
from jax.experimental.pallas import tpu_sc as plsc

_NC = 2
_NS = 16
_L = 16
_NW = _NC * _NS
_RING = 4


def _make_sc_add(B, S, D):
    SPW = S // _NW
    R = 4
    NCHUNK = SPW // R
    NG = D // _L

    def body(x_hbm, pos_hbm, out_hbm, xbuf, pbuf, *sems):
        lsem = sems[:_RING]
        ssem = sems[_RING:]
        wid = lax.axis_index("s") * _NC + lax.axis_index("c")
        base = wid * SPW

        def issue_loads(cc, q):
            row = base + cc * R
            pltpu.async_copy(pos_hbm.at[pl.ds(row, R), :], pbuf.at[q], lsem[q])
            for b in range(B):
                pltpu.async_copy(
                    x_hbm.at[b, pl.ds(row, R), :], xbuf.at[q, b], lsem[q]
                )

        def wait_loads(q):
            pltpu.make_async_copy(
                pos_hbm.at[pl.ds(0, R), :], pbuf.at[q], lsem[q]
            ).wait()
            for b in range(B):
                pltpu.make_async_copy(
                    x_hbm.at[b, pl.ds(0, R), :], xbuf.at[q, b], lsem[q]
                ).wait()

        def issue_stores(cc, q):
            row = base + cc * R
            for b in range(B):
                pltpu.async_copy(
                    xbuf.at[q, b], out_hbm.at[b, pl.ds(row, R), :], ssem[q]
                )

        def wait_stores(q):
            for b in range(B):
                pltpu.make_async_copy(
                    xbuf.at[q, b], out_hbm.at[b, pl.ds(0, R), :], ssem[q]
                ).wait()

        issue_loads(0, 0)

        @pl.loop(0, NCHUNK, step=_RING)
        def _(ci):
            for q in range(_RING):
                cc = ci + q
                nq = (q + 1) % _RING

                @pl.when(cc >= _RING - 1)
                def _():
                    wait_stores(nq)

                @pl.when(cc < NCHUNK - 1)
                def _():
                    issue_loads(cc + 1, nq)

                wait_loads(q)

                @plsc.parallel_loop(0, NG, unroll=4)
                def _(j):
                    ds = pl.ds(j * _L, _L)
                    for r in range(R):
                        pv = pbuf[q, r, ds]
                        for b in range(B):
                            xbuf[q, b, r, ds] = xbuf[q, b, r, ds] + pv

                issue_stores(cc, q)

        for q in ((NCHUNK - 3) % _RING, (NCHUNK - 2) % _RING, (NCHUNK - 1) % _RING):
            wait_stores(q)

    mesh = plsc.VectorSubcoreMesh(core_axis_name="c", subcore_axis_name="s")
    return pl.kernel(
        body,
        out_type=jax.ShapeDtypeStruct((B, S, D), jnp.float32),
        mesh=mesh,
        scratch_types=(
            [
                pltpu.VMEM((_RING, B, R, D), jnp.float32),
                pltpu.VMEM((_RING, R, D), jnp.float32),
            ]
            + [pltpu.SemaphoreType.DMA] * (2 * _RING)
        ),
    )


def kernel(x, position_embeddings):
    B, S, D = x.shape
    pos = position_embeddings[:S]
    return _make_sc_add(B, S, D)(x, pos)
